# baseline (speedup 1.0000x reference)
import functools

import jax
import jax.numpy as jnp
from jax.experimental import pallas as pl

B = 4096
HIDDEN = 512
NUM_OBJECTS = 5
EMBED = 32
FEAT = 25
PATCH_K = 12

BLOCK_B = 256


def _fused_kernel(p_ref, w1_ref, b1_ref, w2_ref, b2_ref, wf1_ref, bf1_ref,
                  wf2_ref, bf2_ref, lng_ref, lnb_ref, wf3_ref, bf3_ref,
                  o_ref):
    bb = p_ref.shape[0] // FEAT

    p = p_ref[...].astype(jnp.bfloat16)
    h1 = jnp.dot(p, w1_ref[...], preferred_element_type=jnp.float32)
    h1 = jnp.maximum(h1 + b1_ref[...], 0.0)

    h2 = jnp.dot(h1.astype(jnp.bfloat16), w2_ref[...],
                 preferred_element_type=jnp.float32)
    h2 = jax.nn.sigmoid(h2 + b2_ref[...])

    hf = jnp.swapaxes(h2.reshape(bb, FEAT, NUM_OBJECTS), 1, 2)
    hf = hf.reshape(bb * NUM_OBJECTS, FEAT)

    x = jnp.dot(hf.astype(jnp.bfloat16), wf1_ref[...],
                preferred_element_type=jnp.float32)
    x = jnp.maximum(x + bf1_ref[...], 0.0)

    x = jnp.dot(x.astype(jnp.bfloat16), wf2_ref[...],
                preferred_element_type=jnp.float32)
    x = x + bf2_ref[...]

    mu = jnp.mean(x, axis=-1, keepdims=True)
    xc = x - mu
    var = jnp.mean(xc * xc, axis=-1, keepdims=True)
    x = xc * jax.lax.rsqrt(var + 1e-5) * lng_ref[...] + lnb_ref[...]
    x = jnp.maximum(x, 0.0)

    out = jnp.dot(x.astype(jnp.bfloat16), wf3_ref[...],
                  preferred_element_type=jnp.float32)
    o_ref[...] = out + bf3_ref[...]


@jax.jit
def kernel(obs, cnn1_w, cnn1_b, bn_gamma, bn_beta, bn_mean, bn_var, cnn2_w,
           cnn2_b, fc1_w, fc1_b, fc2_w, fc2_b, ln_gamma, ln_beta, fc3_w,
           fc3_b):
    f32 = jnp.float32
    bf16 = jnp.bfloat16

    scale = bn_gamma / jnp.sqrt(bn_var + 1e-5)
    w1 = (cnn1_w * scale[:, None, None, None]).reshape(HIDDEN, PATCH_K).T
    b1 = (cnn1_b - bn_mean) * scale + bn_beta
    w2 = cnn2_w.reshape(NUM_OBJECTS, HIDDEN).T
    wf1 = fc1_w.T
    wf2 = fc2_w.T
    wf3 = fc3_w.T

    pat = obs.reshape(B, 3, 5, 10, 5, 10)[:, :, :, :2, :, :2]
    pat = pat.transpose(0, 2, 4, 1, 3, 5).reshape(B * FEAT, PATCH_K)

    grid = (B // BLOCK_B,)
    row = lambda b: (b, 0)
    fixed = lambda b: (0, 0)

    def wspec(a):
        return pl.BlockSpec(a.shape, fixed)

    args = (
        pat,
        w1.astype(bf16), b1.reshape(1, HIDDEN).astype(f32),
        w2.astype(bf16), cnn2_b.reshape(1, NUM_OBJECTS).astype(f32),
        wf1.astype(bf16), fc1_b.reshape(1, HIDDEN).astype(f32),
        wf2.astype(bf16), fc2_b.reshape(1, HIDDEN).astype(f32),
        ln_gamma.reshape(1, HIDDEN).astype(f32),
        ln_beta.reshape(1, HIDDEN).astype(f32),
        wf3.astype(bf16), fc3_b.reshape(1, EMBED).astype(f32),
    )
    in_specs = [pl.BlockSpec((BLOCK_B * FEAT, PATCH_K), row)]
    in_specs += [wspec(a) for a in args[1:]]

    out = pl.pallas_call(
        _fused_kernel,
        grid=grid,
        in_specs=in_specs,
        out_specs=pl.BlockSpec((BLOCK_B * NUM_OBJECTS, EMBED), row),
        out_shape=jax.ShapeDtypeStruct((B * NUM_OBJECTS, EMBED), f32),
    )(*args)
    return out.reshape(B, NUM_OBJECTS, EMBED)


# speedup vs baseline: 1.6416x; 1.6416x over previous
import jax
import jax.numpy as jnp
from jax.experimental import pallas as pl

B = 4096
HIDDEN = 512
NUM_OBJECTS = 5
EMBED = 32
FEAT = 25

BLOCK_B = 256


def _fused_kernel(a_ref, v_ref, b1_ref, w2_ref, b2_ref, wf1_ref, bf1_ref,
                  wf2_ref, bf2_ref, lng_ref, lnb_ref, wf3_ref, bf3_ref,
                  o_ref):
    bb = a_ref.shape[0]

    a = a_ref[...].reshape(bb * 5, 300).astype(jnp.bfloat16)
    h1 = jnp.dot(a, v_ref[...], preferred_element_type=jnp.float32)
    h1 = jnp.maximum(h1 + b1_ref[...], 0.0)

    h2 = jnp.dot(h1.astype(jnp.bfloat16), w2_ref[...],
                 preferred_element_type=jnp.float32)
    h2 = jax.nn.sigmoid(h2 + b2_ref[...])

    xm = jnp.transpose(h2.reshape(bb, 5, 5, 5), (0, 3, 1, 2))
    xm = xm.reshape(bb * NUM_OBJECTS, FEAT)

    x = jnp.dot(xm.astype(jnp.bfloat16), wf1_ref[...],
                preferred_element_type=jnp.float32)
    x = jnp.maximum(x + bf1_ref[...], 0.0)

    x = jnp.dot(x.astype(jnp.bfloat16), wf2_ref[...],
                preferred_element_type=jnp.float32)
    x = x + bf2_ref[...]

    mu = jnp.mean(x, axis=-1, keepdims=True)
    xc = x - mu
    var = jnp.mean(xc * xc, axis=-1, keepdims=True)
    x = xc * jax.lax.rsqrt(var + 1e-5) * lng_ref[...] + lnb_ref[...]
    x = jnp.maximum(x, 0.0)

    out = jnp.dot(x.astype(jnp.bfloat16), wf3_ref[...],
                  preferred_element_type=jnp.float32)
    o_ref[...] = out + bf3_ref[...]


@jax.jit
def kernel(obs, cnn1_w, cnn1_b, bn_gamma, bn_beta, bn_mean, bn_var, cnn2_w,
           cnn2_b, fc1_w, fc1_b, fc2_w, fc2_b, ln_gamma, ln_beta, fc3_w,
           fc3_b):
    f32 = jnp.float32
    bf16 = jnp.bfloat16

    scale = bn_gamma / jnp.sqrt(bn_var + 1e-5)
    w1f = cnn1_w * scale[:, None, None, None]
    b1 = (cnn1_b - bn_mean) * scale + bn_beta

    wpad = jnp.zeros((2, 3, 10, HIDDEN), f32)
    wpad = wpad.at[:, :, :2, :].set(w1f.transpose(2, 1, 3, 0))
    m2 = jnp.eye(5, dtype=f32)
    v6 = wpad[:, :, None, :, None, :] * m2[None, None, :, None, :, None]
    v = v6.reshape(300, 5 * HIDDEN)
    b1bd = jnp.tile(b1, 5)

    w2 = cnn2_w.reshape(NUM_OBJECTS, HIDDEN).T
    w2bd = jnp.kron(jnp.eye(5, dtype=f32), w2)
    b2bd = jnp.tile(cnn2_b, 5)

    wf1 = fc1_w.T
    wf2 = fc2_w.T
    wf3 = fc3_w.T

    pat = obs.reshape(B, 3, 5, 10, 50)[:, :, :, :2, :]
    pat = pat.transpose(0, 2, 3, 1, 4).reshape(B, 5, 300)

    grid = (B // BLOCK_B,)
    row = lambda b: (b, 0, 0)
    row2 = lambda b: (b, 0)
    fixed = lambda b: (0, 0)

    def wspec(a):
        return pl.BlockSpec(a.shape, fixed)

    args = (
        pat,
        v.astype(bf16), b1bd.reshape(1, 5 * HIDDEN).astype(f32),
        w2bd.astype(bf16), b2bd.reshape(1, FEAT).astype(f32),
        wf1.astype(bf16), fc1_b.reshape(1, HIDDEN).astype(f32),
        wf2.astype(bf16), fc2_b.reshape(1, HIDDEN).astype(f32),
        ln_gamma.reshape(1, HIDDEN).astype(f32),
        ln_beta.reshape(1, HIDDEN).astype(f32),
        wf3.astype(bf16), fc3_b.reshape(1, EMBED).astype(f32),
    )
    in_specs = [pl.BlockSpec((BLOCK_B, 5, 300), row)]
    in_specs += [wspec(a) for a in args[1:]]

    out = pl.pallas_call(
        _fused_kernel,
        grid=grid,
        in_specs=in_specs,
        out_specs=pl.BlockSpec((BLOCK_B * NUM_OBJECTS, EMBED), row2),
        out_shape=jax.ShapeDtypeStruct((B * NUM_OBJECTS, EMBED), f32),
    )(*args)
    return out.reshape(B, NUM_OBJECTS, EMBED)
